# baseline (speedup 1.0000x reference)
import functools

import jax
import jax.numpy as jnp
from jax import lax
from jax.experimental import pallas as pl
from jax.experimental.pallas import tpu as pltpu
from jax.experimental.pallas import tpu_sc as plsc

_B, _C, _H, _W = 8, 3, 512, 512
_HW = _H * _W
_N = _B * _HW
_NW = 32
_PER_TILE = _N // _NW
_TILES_PER_IMG = _HW // _PER_TILE
_P = 4096
_CHUNKS = _PER_TILE // _P
_VECS = _P // 16

_MIN_DEPTH = 0.1
_MAX_DEPTH = 10.0


def _sc_body(depth_hbm, rgb_hbm, idx_hbm, val_hbm,
             dv, rv, gv, bv, oi, ov):
    nc = 2
    wid = lax.axis_index("s") * nc + lax.axis_index("c")
    b_id = wid // _TILES_PER_IMG
    q = wid % _TILES_PER_IMG
    gbase = wid * _PER_TILE
    pixbase = q * _PER_TILE

    lane = lax.broadcasted_iota(jnp.int32, (16,), 0)
    lane4 = lane * 4
    lane3 = lane * 3
    lanef = lane.astype(jnp.float32)
    neg1 = jnp.full((16,), -1, jnp.int32)
    zero3 = jnp.zeros((16,), jnp.float32)
    bvecv = jnp.full((16,), 1, jnp.int32) * b_id

    for k in range(_CHUNKS):
        goff = gbase + k * _P
        poff = pixbase + k * _P
        pltpu.sync_copy(depth_hbm.at[pl.ds(goff, _P)], dv)
        roff = (b_id * 3) * _HW + poff
        pltpu.sync_copy(rgb_hbm.at[pl.ds(roff, _P)], rv)
        pltpu.sync_copy(rgb_hbm.at[pl.ds(roff + _HW, _P)], gv)
        pltpu.sync_copy(rgb_hbm.at[pl.ds(roff + 2 * _HW, _P)], bv)

        def body(i, carry):
            off = i * 16
            d = dv[pl.ds(off, 16)]
            r = rv[pl.ds(off, 16)]
            g = gv[pl.ds(off, 16)]
            b = bv[pl.ds(off, 16)]
            wbase = lax.rem(off, 512)
            h = (poff + off) // 512
            wf = lanef + wbase.astype(jnp.float32)
            hf = jnp.full((16,), 1.0, jnp.float32) * h.astype(jnp.float32)
            xs = (wf - 256.0) * d
            ys = (hf - 256.0) * d
            tx = xs * 0.125 + 32.0
            ty = ys * 0.125 + 32.0
            tz = d * 32.0 + 32.0
            ix = jnp.minimum(jnp.maximum(tx.astype(jnp.int32), 0), 63)
            iy = jnp.minimum(jnp.maximum(ty.astype(jnp.int32), 0), 63)
            iz = jnp.minimum(jnp.maximum(tz.astype(jnp.int32), 0), 63)
            valid = (d > _MIN_DEPTH) & (d < _MAX_DEPTH)
            ib = jnp.where(valid, bvecv, neg1)
            ix = jnp.where(valid, ix, neg1)
            iy = jnp.where(valid, iy, neg1)
            iz = jnp.where(valid, iz, neg1)
            rm = jnp.where(valid, r, zero3)
            gm = jnp.where(valid, g, zero3)
            bm = jnp.where(valid, b, zero3)
            i4 = lane4 + off * 4
            plsc.store_scatter(oi, [i4], ib)
            plsc.store_scatter(oi, [i4 + 1], ix)
            plsc.store_scatter(oi, [i4 + 2], iy)
            plsc.store_scatter(oi, [i4 + 3], iz)
            i3 = lane3 + off * 3
            plsc.store_scatter(ov, [i3], rm)
            plsc.store_scatter(ov, [i3 + 1], gm)
            plsc.store_scatter(ov, [i3 + 2], bm)
            return carry

        lax.fori_loop(0, _VECS, body, 0)

        pltpu.sync_copy(oi, idx_hbm.at[pl.ds(goff * 4, _P * 4)])
        pltpu.sync_copy(ov, val_hbm.at[pl.ds(goff * 3, _P * 3)])


@jax.jit
def _convert(depth_flat, rgb_flat):
    mesh = plsc.VectorSubcoreMesh(core_axis_name="c", subcore_axis_name="s")
    f = functools.partial(
        pl.kernel,
        mesh=mesh,
        compiler_params=pltpu.CompilerParams(needs_layout_passes=False),
        out_type=[
            jax.ShapeDtypeStruct((_N * 4,), jnp.int32),
            jax.ShapeDtypeStruct((_N * 3,), jnp.float32),
        ],
        scratch_types=[
            pltpu.VMEM((_P,), jnp.float32),
            pltpu.VMEM((_P,), jnp.float32),
            pltpu.VMEM((_P,), jnp.float32),
            pltpu.VMEM((_P,), jnp.float32),
            pltpu.VMEM((_P * 4,), jnp.int32),
            pltpu.VMEM((_P * 3,), jnp.float32),
        ],
    )(_sc_body)
    return f(depth_flat, rgb_flat)


def kernel(rgb, depth):
    depth_flat = depth.reshape(_N)
    rgb_flat = rgb.reshape(_B * _C * _HW)
    idx_flat, val_flat = _convert(depth_flat, rgb_flat)
    return idx_flat.reshape(_N, 4), val_flat.reshape(_N, 3)


# speedup vs baseline: 14.9490x; 14.9490x over previous
import functools

import jax
import jax.numpy as jnp
from jax import lax
from jax.experimental import pallas as pl
from jax.experimental.pallas import tpu as pltpu
from jax.experimental.pallas import tpu_sc as plsc

_B, _C, _H, _W = 8, 3, 512, 512
_HW = _H * _W
_N = _B * _HW
_NW = 32
_BANDS = _B * (_H // 8)
_BPW = _BANDS // _NW
_PB = 8 * _W
_OB = 4 * _PB

_MIN_DEPTH = 0.1
_MAX_DEPTH = 10.0


def _sc_body(depth_hbm, rgb_hbm, idx_hbm, val_hbm, dv, rv, gv, bv, oi, ov):
    nc = 2
    wid = lax.axis_index("s") * nc + lax.axis_index("c")

    lane = lax.broadcasted_iota(jnp.int32, (16,), 0)
    lanef = lane.astype(jnp.float32)
    neg1 = jnp.full((16,), -1, jnp.int32)
    zerov = jnp.zeros((16,), jnp.float32)

    def band_body(bi, carry):
        t = wid * _BPW + bi
        b_id = t >> 6
        rt = t & 63
        doff = t * _PB
        roff = (b_id * 3 * 64 + rt) * _PB
        pltpu.sync_copy(depth_hbm.at[pl.ds(doff, _PB)], dv)
        pltpu.sync_copy(rgb_hbm.at[pl.ds(roff, _PB)], rv)
        pltpu.sync_copy(rgb_hbm.at[pl.ds(roff + 64 * _PB, _PB)], gv)
        pltpu.sync_copy(rgb_hbm.at[pl.ds(roff + 128 * _PB, _PB)], bv)

        bvecv = jnp.full((16,), 1, jnp.int32) * b_id

        def body(v, c2):
            r = v >> 5
            ct = (v >> 3) & 3
            k = v & 7
            ob = ((v >> 3) << 9) + (k << 4)
            io = (ct << 10) + (r << 7) + (k << 4)
            d = dv[pl.ds(io, 16)]
            rr = rv[pl.ds(io, 16)]
            gg = gv[pl.ds(io, 16)]
            bb = bv[pl.ds(io, 16)]
            wadj = (ct << 7) + (k << 4) - 256
            wf = lanef + wadj.astype(jnp.float32)
            h256 = ((rt << 3) + r - 256).astype(jnp.float32)
            tx = (wf * d) * 0.125 + 32.0
            ty = (d * h256) * 0.125 + 32.0
            tz = d * 32.0 + 32.0
            ix = jnp.minimum(jnp.maximum(tx.astype(jnp.int32), 0), 63)
            iy = jnp.minimum(jnp.maximum(ty.astype(jnp.int32), 0), 63)
            iz = jnp.minimum(jnp.maximum(tz.astype(jnp.int32), 0), 63)
            valid = (d > _MIN_DEPTH) & (d < _MAX_DEPTH)
            oi[pl.ds(ob, 16)] = jnp.where(valid, bvecv, neg1)
            oi[pl.ds(ob + 128, 16)] = jnp.where(valid, ix, neg1)
            oi[pl.ds(ob + 256, 16)] = jnp.where(valid, iy, neg1)
            oi[pl.ds(ob + 384, 16)] = jnp.where(valid, iz, neg1)
            ov[pl.ds(ob, 16)] = jnp.where(valid, rr, zerov)
            ov[pl.ds(ob + 128, 16)] = jnp.where(valid, gg, zerov)
            ov[pl.ds(ob + 256, 16)] = jnp.where(valid, bb, zerov)
            return c2

        lax.fori_loop(0, _PB // 16, body, 0, unroll=4)

        obase = (b_id * 2048 + rt * 32) * 512
        pltpu.sync_copy(oi, idx_hbm.at[pl.ds(obase, _OB)])
        pltpu.sync_copy(ov, val_hbm.at[pl.ds(obase, _OB)])
        return carry

    lax.fori_loop(0, _BPW, band_body, 0)


@jax.jit
def _convert(depth_p, rgb_p):
    mesh = plsc.VectorSubcoreMesh(core_axis_name="c", subcore_axis_name="s")
    f = functools.partial(
        pl.kernel,
        mesh=mesh,
        compiler_params=pltpu.CompilerParams(needs_layout_passes=False),
        out_type=[
            jax.ShapeDtypeStruct((_N * 4,), jnp.int32),
            jax.ShapeDtypeStruct((_N * 4,), jnp.float32),
        ],
        scratch_types=[
            pltpu.VMEM((_PB,), jnp.float32),
            pltpu.VMEM((_PB,), jnp.float32),
            pltpu.VMEM((_PB,), jnp.float32),
            pltpu.VMEM((_PB,), jnp.float32),
            pltpu.VMEM((_OB,), jnp.int32),
            pltpu.VMEM((_OB,), jnp.float32),
        ],
    )(_sc_body)
    return f(depth_p, rgb_p)


def kernel(rgb, depth):
    rgb_p = (rgb.reshape(_B, _C, 64, 8, 4, 128)
             .transpose(0, 1, 2, 4, 3, 5).reshape(-1))
    dep_p = (depth.reshape(_B, 64, 8, 4, 128)
             .transpose(0, 1, 3, 2, 4).reshape(-1))
    oi1, ov1 = _convert(dep_p, rgb_p)
    idx = oi1.reshape(_N // 128, 4, 128).transpose(0, 2, 1).reshape(_N, 4)
    val = (ov1.reshape(_N // 128, 4, 128)[:, :3, :]
           .transpose(0, 2, 1).reshape(_N, 3))
    return idx, val


# speedup vs baseline: 22.9893x; 1.5378x over previous
import functools

import jax
import jax.numpy as jnp
from jax import lax
from jax.experimental import pallas as pl
from jax.experimental.pallas import tpu as pltpu
from jax.experimental.pallas import tpu_sc as plsc

_B, _C, _H, _W = 8, 3, 512, 512
_HW = _H * _W
_N = _B * _HW
_NW = 32
_BANDS = _B * (_H // 8)
_BPW = _BANDS // _NW
_PB = 8 * _W
_OB = 4 * _PB

_MIN_DEPTH = 0.1
_MAX_DEPTH = 10.0


def _sc_body(depth_hbm, rgb_hbm, idx_hbm, val_hbm,
             dv0, dv1, rv0, rv1, gv0, gv1, bv0, bv1,
             oi0, oi1, ov0, ov1, isem0, isem1, osem0, osem1):
    nc = 2
    wid = lax.axis_index("s") * nc + lax.axis_index("c")
    dv = (dv0, dv1)
    rv = (rv0, rv1)
    gv = (gv0, gv1)
    bv = (bv0, bv1)
    oi = (oi0, oi1)
    ov = (ov0, ov1)
    isem = (isem0, isem1)
    osem = (osem0, osem1)

    lane = lax.broadcasted_iota(jnp.int32, (16,), 0)
    lanef = lane.astype(jnp.float32)
    neg1 = jnp.full((16,), -1, jnp.int32)
    zerov = jnp.zeros((16,), jnp.float32)

    def start_in(bi, s):
        t = wid * _BPW + bi
        b_id = t >> 6
        rt = t & 63
        doff = t * _PB
        roff = (b_id * 3 * 64 + rt) * _PB
        return (
            pltpu.async_copy(depth_hbm.at[pl.ds(doff, _PB)], dv[s], isem[s]),
            pltpu.async_copy(rgb_hbm.at[pl.ds(roff, _PB)], rv[s], isem[s]),
            pltpu.async_copy(rgb_hbm.at[pl.ds(roff + 64 * _PB, _PB)],
                             gv[s], isem[s]),
            pltpu.async_copy(rgb_hbm.at[pl.ds(roff + 128 * _PB, _PB)],
                             bv[s], isem[s]),
        )

    def start_out(bi, s):
        t = wid * _BPW + bi
        b_id = t >> 6
        rt = t & 63
        obase = (b_id * 2048 + rt * 32) * 512
        return (
            pltpu.async_copy(oi[s], idx_hbm.at[pl.ds(obase, _OB)], osem[s]),
            pltpu.async_copy(ov[s], val_hbm.at[pl.ds(obase, _OB)], osem[s]),
        )

    def compute(bi, s):
        t = wid * _BPW + bi
        b_id = t >> 6
        rt = t & 63
        bvecv = jnp.full((16,), 1, jnp.int32) * b_id
        dvs, rvs, gvs, bvs, ois, ovs = (
            dv[s], rv[s], gv[s], bv[s], oi[s], ov[s])

        def body(v, c2):
            r = v >> 5
            ct = (v >> 3) & 3
            k = v & 7
            ob = ((v >> 3) << 9) + (k << 4)
            io = (ct << 10) + (r << 7) + (k << 4)
            d = dvs[pl.ds(io, 16)]
            rr = rvs[pl.ds(io, 16)]
            gg = gvs[pl.ds(io, 16)]
            bb = bvs[pl.ds(io, 16)]
            wadj = (ct << 7) + (k << 4) - 256
            wf = lanef + wadj.astype(jnp.float32)
            h256 = ((rt << 3) + r - 256).astype(jnp.float32)
            tx = (wf * d) * 0.125 + 32.0
            ty = (d * h256) * 0.125 + 32.0
            tz = d * 32.0 + 32.0
            ix = jnp.minimum(jnp.maximum(tx.astype(jnp.int32), 0), 63)
            iy = jnp.minimum(jnp.maximum(ty.astype(jnp.int32), 0), 63)
            iz = jnp.minimum(jnp.maximum(tz.astype(jnp.int32), 0), 63)
            valid = (d > _MIN_DEPTH) & (d < _MAX_DEPTH)
            ois[pl.ds(ob, 16)] = jnp.where(valid, bvecv, neg1)
            ois[pl.ds(ob + 128, 16)] = jnp.where(valid, ix, neg1)
            ois[pl.ds(ob + 256, 16)] = jnp.where(valid, iy, neg1)
            ois[pl.ds(ob + 384, 16)] = jnp.where(valid, iz, neg1)
            ovs[pl.ds(ob, 16)] = jnp.where(valid, rr, zerov)
            ovs[pl.ds(ob + 128, 16)] = jnp.where(valid, gg, zerov)
            ovs[pl.ds(ob + 256, 16)] = jnp.where(valid, bb, zerov)
            return c2

        lax.fori_loop(0, _PB // 16, body, 0, unroll=4)

    in_cp = {0: start_in(0, 0)}
    out_cp = {}
    for bi in range(_BPW):
        s = bi % 2
        for c in in_cp.pop(bi):
            c.wait()
        if bi + 1 < _BPW:
            in_cp[bi + 1] = start_in(bi + 1, 1 - s)
        if bi - 2 in out_cp:
            for c in out_cp.pop(bi - 2):
                c.wait()
        compute(bi, s)
        out_cp[bi] = start_out(bi, s)
    for cps in out_cp.values():
        for c in cps:
            c.wait()


@jax.jit
def _convert(depth_p, rgb_p):
    mesh = plsc.VectorSubcoreMesh(core_axis_name="c", subcore_axis_name="s")
    f = functools.partial(
        pl.kernel,
        mesh=mesh,
        compiler_params=pltpu.CompilerParams(needs_layout_passes=False),
        out_type=[
            jax.ShapeDtypeStruct((_N * 4,), jnp.int32),
            jax.ShapeDtypeStruct((_N * 4,), jnp.float32),
        ],
        scratch_types=(
            [pltpu.VMEM((_PB,), jnp.float32)] * 8
            + [pltpu.VMEM((_OB,), jnp.int32)] * 2
            + [pltpu.VMEM((_OB,), jnp.float32)] * 2
            + [pltpu.SemaphoreType.DMA] * 4
        ),
    )(_sc_body)
    return f(depth_p, rgb_p)


def kernel(rgb, depth):
    rgb_p = (rgb.reshape(_B, _C, 64, 8, 4, 128)
             .transpose(0, 1, 2, 4, 3, 5).reshape(-1))
    dep_p = (depth.reshape(_B, 64, 8, 4, 128)
             .transpose(0, 1, 3, 2, 4).reshape(-1))
    oi1, ov1 = _convert(dep_p, rgb_p)
    idx = oi1.reshape(_N // 128, 4, 128).transpose(0, 2, 1).reshape(_N, 4)
    val = (ov1.reshape(_N // 128, 4, 128)[:, :3, :]
           .transpose(0, 2, 1).reshape(_N, 3))
    return idx, val
